# Initial kernel scaffold; baseline (speedup 1.0000x reference)
import functools

import numpy as np
import jax
import jax.numpy as jnp
from jax import lax
from jax.experimental import pallas as pl
from jax.experimental.pallas import tpu as pltpu
from jax.experimental.pallas import tpu_sc as plsc

B = 1024
MAX_OUT = 200
REL_DIM = 64
STATE_DIM = 128
PAD_ID = 0

NW = 32
CHUNK = 128
N_IDS = B + B + B * MAX_OUT
CH = -(-N_IDS // (NW * CHUNK))
N_PAD = NW * CH * CHUNK

BB = 64


def _uniform_constant():
    def rotl(x, d):
        return (x << np.uint32(d)) | (x >> np.uint32(32 - d))

    with np.errstate(over="ignore"):
        n = np.arange(B * MAX_OUT, dtype=np.uint32)
        ks = [np.uint32(0), np.uint32(1234),
              np.uint32(0) ^ np.uint32(1234) ^ np.uint32(0x1BD11BDA)]
        x = [np.zeros_like(n) + ks[0], n + ks[1]]
        rot0, rot1 = (13, 15, 26, 6), (17, 29, 16, 24)

        def rounds(x, rots):
            for r in rots:
                x[0] = x[0] + x[1]
                x[1] = x[0] ^ rotl(x[1], r)
            return x

        x = rounds(x, rot0); x[0] = x[0] + ks[1]; x[1] = x[1] + ks[2] + np.uint32(1)
        x = rounds(x, rot1); x[0] = x[0] + ks[2]; x[1] = x[1] + ks[0] + np.uint32(2)
        x = rounds(x, rot0); x[0] = x[0] + ks[0]; x[1] = x[1] + ks[1] + np.uint32(3)
        x = rounds(x, rot1); x[0] = x[0] + ks[1]; x[1] = x[1] + ks[2] + np.uint32(4)
        x = rounds(x, rot0); x[0] = x[0] + ks[2]; x[1] = x[1] + ks[0] + np.uint32(5)
        bits = x[0] ^ x[1]

    fb = (bits >> np.uint32(9)) | np.uint32(0x3F800000)
    u = fb.view(np.float32) - np.float32(1.0)
    tiny = np.float32(np.finfo(np.float32).tiny)
    u = np.maximum(tiny, (u * np.float32(1.0 - tiny) + tiny).astype(np.float32))
    return u.reshape(B, MAX_OUT)


_UNIFORM = _uniform_constant()


def _sc_gather(table, idx3):
    mesh = plsc.VectorSubcoreMesh(core_axis_name="c", subcore_axis_name="s")

    @functools.partial(
        pl.kernel,
        out_type=jax.ShapeDtypeStruct((N_PAD, REL_DIM), jnp.float32),
        mesh=mesh,
        scratch_types=[
            pltpu.VMEM((CH, CHUNK), jnp.int32),
            pltpu.VMEM((CHUNK, REL_DIM), jnp.float32),
            pltpu.SemaphoreType.DMA,
        ],
    )
    def k(table_hbm, idx_hbm, out_hbm, idx_v, buf, sem):
        wid = lax.axis_index("s") * 2 + lax.axis_index("c")
        pltpu.sync_copy(idx_hbm.at[wid], idx_v)
        base = wid * (CH * CHUNK)

        def body(j, carry):
            pltpu.async_copy(table_hbm.at[idx_v.at[j]], buf, sem).wait()
            pltpu.sync_copy(buf, out_hbm.at[pl.ds(base + j * CHUNK, CHUNK)])
            return carry

        lax.fori_loop(0, CH, body, 0)

    return k(table, idx3)


def _dense_body(emb_ref, h_ref, c_ref, q_ref, wih_ref, whh_ref, bih_ref,
                bhh_ref, w1_ref, b1_ref, w2_ref, b2_ref,
                hout_ref, cout_ref, mlp_ref):
    dn = (((1,), (1,)), ((), ()))
    gates = lax.dot_general(emb_ref[...], wih_ref[...], dn,
                            preferred_element_type=jnp.float32)
    gates = gates + lax.dot_general(h_ref[...], whh_ref[...], dn,
                                    preferred_element_type=jnp.float32)
    gates = gates + bih_ref[...]
    gates = gates + bhh_ref[...]
    i = jax.nn.sigmoid(gates[:, :STATE_DIM])
    f = jax.nn.sigmoid(gates[:, STATE_DIM:2 * STATE_DIM])
    g = jnp.tanh(gates[:, 2 * STATE_DIM:3 * STATE_DIM])
    o = jax.nn.sigmoid(gates[:, 3 * STATE_DIM:])
    c_new = f * c_ref[...] + i * g
    h_new = o * jnp.tanh(c_new)
    hout_ref[...] = h_new
    cout_ref[...] = c_new
    sq = jnp.concatenate([h_new, q_ref[...]], axis=1)
    hid = jnp.maximum(
        lax.dot_general(sq, w1_ref[...], dn,
                        preferred_element_type=jnp.float32) + b1_ref[...], 0.0)
    mlp = jnp.maximum(
        lax.dot_general(hid, w2_ref[...], dn,
                        preferred_element_type=jnp.float32) + b2_ref[...], 0.0)
    mlp_ref[...] = mlp


def _dense(prev_emb, h, c, q_emb, W_ih, W_hh, b_ih, b_hh, W1, b1, W2, b2):
    return pl.pallas_call(
        _dense_body,
        out_shape=(
            jax.ShapeDtypeStruct((B, STATE_DIM), jnp.float32),
            jax.ShapeDtypeStruct((B, STATE_DIM), jnp.float32),
            jax.ShapeDtypeStruct((B, REL_DIM), jnp.float32),
        ),
    )(prev_emb, h, c, q_emb, W_ih, W_hh, b_ih.reshape(1, -1),
      b_hh.reshape(1, -1), W1, b1.reshape(1, -1), W2, b2.reshape(1, -1))


def _score_body(rows_ref, ids_ref, mlp_ref, u_ref,
                logits_ref, loss_ref, act_ref, cho_ref):
    rows = rows_ref[...]
    mlp = mlp_ref[...]
    scores = jnp.sum(rows * mlp[:, None, :], axis=-1)
    ids = ids_ref[...]
    masked = jnp.where(ids == PAD_ID, jnp.float32(-99999.0), scores)
    gum = -jnp.log(-jnp.log(u_ref[...]))
    z = gum + masked
    zmax = jnp.max(z, axis=1, keepdims=True)
    miota = lax.broadcasted_iota(jnp.int32, (BB, MAX_OUT), 1)
    act = jnp.min(jnp.where(z == zmax, miota, MAX_OUT), axis=1, keepdims=True)
    act_ref[...] = act
    cho_ref[...] = jnp.sum(jnp.where(miota == act, ids, 0),
                           axis=1, keepdims=True)
    smax = jnp.max(masked, axis=1, keepdims=True)
    sh = masked - smax
    lse = jnp.log(jnp.sum(jnp.exp(sh), axis=1, keepdims=True))
    lg = sh - lse
    logits_ref[...] = lg
    loss_ref[...] = -jnp.sum(jnp.where(miota == act, lg, 0.0),
                             axis=1, keepdims=True)


def _score(rows, ids, mlp, u):
    grid = (B // BB,)
    return pl.pallas_call(
        _score_body,
        grid=grid,
        in_specs=[
            pl.BlockSpec((BB, MAX_OUT, REL_DIM), lambda i: (i, 0, 0)),
            pl.BlockSpec((BB, MAX_OUT), lambda i: (i, 0)),
            pl.BlockSpec((BB, REL_DIM), lambda i: (i, 0)),
            pl.BlockSpec((BB, MAX_OUT), lambda i: (i, 0)),
        ],
        out_specs=[
            pl.BlockSpec((BB, MAX_OUT), lambda i: (i, 0)),
            pl.BlockSpec((BB, 1), lambda i: (i, 0)),
            pl.BlockSpec((BB, 1), lambda i: (i, 0)),
            pl.BlockSpec((BB, 1), lambda i: (i, 0)),
        ],
        out_shape=(
            jax.ShapeDtypeStruct((B, MAX_OUT), jnp.float32),
            jax.ShapeDtypeStruct((B, 1), jnp.float32),
            jax.ShapeDtypeStruct((B, 1), jnp.int32),
            jax.ShapeDtypeStruct((B, 1), jnp.int32),
        ),
    )(rows, ids, mlp, u)


def kernel(prev_state_h, prev_state_c, prev_relation, actions_id, queries,
           rel_emb, W_ih, W_hh, b_ih, b_hh, W1, b1, W2, b2):
    out_ids = actions_id[:, :, 0]
    ids = jnp.concatenate([
        prev_relation.astype(jnp.int32),
        queries.astype(jnp.int32),
        out_ids.reshape(-1),
        jnp.zeros((N_PAD - N_IDS,), jnp.int32),
    ])
    rows = _sc_gather(rel_emb, ids.reshape(NW, CH, CHUNK))
    prev_emb = rows[:B]
    q_emb = rows[B:2 * B]
    gathered = rows[2 * B:2 * B + B * MAX_OUT].reshape(B, MAX_OUT, REL_DIM)

    h_new, c_new, mlp = _dense(prev_emb, prev_state_h, prev_state_c, q_emb,
                               W_ih, W_hh, b_ih, b_hh, W1, b1, W2, b2)
    u = jnp.asarray(_UNIFORM)
    logits, loss2, act2, cho2 = _score(gathered, out_ids, mlp, u)
    return (loss2[:, 0], logits, h_new, c_new, act2[:, 0], cho2[:, 0])


# speedup vs baseline: 2.0727x; 2.0727x over previous
import functools

import numpy as np
import jax
import jax.numpy as jnp
from jax import lax
from jax.experimental import pallas as pl
from jax.experimental.pallas import tpu as pltpu
from jax.experimental.pallas import tpu_sc as plsc

B = 1024
MAX_OUT = 200
REL_DIM = 64
STATE_DIM = 128
PAD_ID = 0

NW = 32
CHUNK = 128
N_IDS = B + B + B * MAX_OUT
CH = -(-N_IDS // (NW * CHUNK))
N_PAD = NW * CH * CHUNK

BB = 64


def _uniform_constant():
    def rotl(x, d):
        return (x << np.uint32(d)) | (x >> np.uint32(32 - d))

    with np.errstate(over="ignore"):
        n = np.arange(B * MAX_OUT, dtype=np.uint32)
        ks = [np.uint32(0), np.uint32(1234),
              np.uint32(0) ^ np.uint32(1234) ^ np.uint32(0x1BD11BDA)]
        x = [np.zeros_like(n) + ks[0], n + ks[1]]
        rot0, rot1 = (13, 15, 26, 6), (17, 29, 16, 24)

        def rounds(x, rots):
            for r in rots:
                x[0] = x[0] + x[1]
                x[1] = x[0] ^ rotl(x[1], r)
            return x

        x = rounds(x, rot0); x[0] = x[0] + ks[1]; x[1] = x[1] + ks[2] + np.uint32(1)
        x = rounds(x, rot1); x[0] = x[0] + ks[2]; x[1] = x[1] + ks[0] + np.uint32(2)
        x = rounds(x, rot0); x[0] = x[0] + ks[0]; x[1] = x[1] + ks[1] + np.uint32(3)
        x = rounds(x, rot1); x[0] = x[0] + ks[1]; x[1] = x[1] + ks[2] + np.uint32(4)
        x = rounds(x, rot0); x[0] = x[0] + ks[2]; x[1] = x[1] + ks[0] + np.uint32(5)
        bits = x[0] ^ x[1]

    fb = (bits >> np.uint32(9)) | np.uint32(0x3F800000)
    u = fb.view(np.float32) - np.float32(1.0)
    tiny = np.float32(np.finfo(np.float32).tiny)
    u = np.maximum(tiny, (u * np.float32(1.0 - tiny) + tiny).astype(np.float32))
    return u.reshape(B, MAX_OUT)


_UNIFORM = _uniform_constant()


def _sc_gather(table, idx3):
    mesh = plsc.VectorSubcoreMesh(core_axis_name="c", subcore_axis_name="s")

    @functools.partial(
        pl.kernel,
        out_type=jax.ShapeDtypeStruct((N_PAD, REL_DIM), jnp.float32),
        mesh=mesh,
        compiler_params=pltpu.CompilerParams(use_tc_tiling_on_sc=False),
        scratch_types=[
            pltpu.VMEM((CH, CHUNK), jnp.int32),
            pltpu.VMEM((CHUNK, REL_DIM), jnp.float32),
            pltpu.SemaphoreType.DMA,
        ],
    )
    def k(table_hbm, idx_hbm, out_hbm, idx_v, buf, sem):
        wid = lax.axis_index("s") * 2 + lax.axis_index("c")
        pltpu.sync_copy(idx_hbm.at[wid], idx_v)
        base = wid * (CH * CHUNK)

        def body(j, carry):
            pltpu.async_copy(table_hbm.at[idx_v.at[j]], buf, sem).wait()
            pltpu.sync_copy(buf, out_hbm.at[pl.ds(base + j * CHUNK, CHUNK)])
            return carry

        lax.fori_loop(0, CH, body, 0)

    return k(table, idx3)


def _dense_body(emb_ref, h_ref, c_ref, q_ref, wih_ref, whh_ref, bih_ref,
                bhh_ref, w1_ref, b1_ref, w2_ref, b2_ref,
                hout_ref, cout_ref, mlp_ref):
    dn = (((1,), (1,)), ((), ()))
    gates = lax.dot_general(emb_ref[...], wih_ref[...], dn,
                            preferred_element_type=jnp.float32)
    gates = gates + lax.dot_general(h_ref[...], whh_ref[...], dn,
                                    preferred_element_type=jnp.float32)
    gates = gates + bih_ref[...]
    gates = gates + bhh_ref[...]
    i = jax.nn.sigmoid(gates[:, :STATE_DIM])
    f = jax.nn.sigmoid(gates[:, STATE_DIM:2 * STATE_DIM])
    g = jnp.tanh(gates[:, 2 * STATE_DIM:3 * STATE_DIM])
    o = jax.nn.sigmoid(gates[:, 3 * STATE_DIM:])
    c_new = f * c_ref[...] + i * g
    h_new = o * jnp.tanh(c_new)
    hout_ref[...] = h_new
    cout_ref[...] = c_new
    sq = jnp.concatenate([h_new, q_ref[...]], axis=1)
    hid = jnp.maximum(
        lax.dot_general(sq, w1_ref[...], dn,
                        preferred_element_type=jnp.float32) + b1_ref[...], 0.0)
    mlp = jnp.maximum(
        lax.dot_general(hid, w2_ref[...], dn,
                        preferred_element_type=jnp.float32) + b2_ref[...], 0.0)
    mlp_ref[...] = mlp


def _dense(prev_emb, h, c, q_emb, W_ih, W_hh, b_ih, b_hh, W1, b1, W2, b2):
    return pl.pallas_call(
        _dense_body,
        out_shape=(
            jax.ShapeDtypeStruct((B, STATE_DIM), jnp.float32),
            jax.ShapeDtypeStruct((B, STATE_DIM), jnp.float32),
            jax.ShapeDtypeStruct((B, REL_DIM), jnp.float32),
        ),
    )(prev_emb, h, c, q_emb, W_ih, W_hh, b_ih.reshape(1, -1),
      b_hh.reshape(1, -1), W1, b1.reshape(1, -1), W2, b2.reshape(1, -1))


def _score_body(rows_ref, ids_ref, mlp_ref, u_ref,
                logits_ref, loss_ref, act_ref, cho_ref):
    rows = rows_ref[...]
    mlp = mlp_ref[...]
    scores = jnp.sum(rows * mlp[:, None, :], axis=-1)
    ids = ids_ref[...]
    masked = jnp.where(ids == PAD_ID, jnp.float32(-99999.0), scores)
    gum = -jnp.log(-jnp.log(u_ref[...]))
    z = gum + masked
    zmax = jnp.max(z, axis=1, keepdims=True)
    miota = lax.broadcasted_iota(jnp.int32, (BB, MAX_OUT), 1)
    act = jnp.min(jnp.where(z == zmax, miota, MAX_OUT), axis=1, keepdims=True)
    act_ref[...] = act
    cho_ref[...] = jnp.sum(jnp.where(miota == act, ids, 0),
                           axis=1, keepdims=True)
    smax = jnp.max(masked, axis=1, keepdims=True)
    sh = masked - smax
    lse = jnp.log(jnp.sum(jnp.exp(sh), axis=1, keepdims=True))
    lg = sh - lse
    logits_ref[...] = lg
    loss_ref[...] = -jnp.sum(jnp.where(miota == act, lg, 0.0),
                             axis=1, keepdims=True)


def _score(rows, ids, mlp, u):
    grid = (B // BB,)
    return pl.pallas_call(
        _score_body,
        grid=grid,
        in_specs=[
            pl.BlockSpec((BB, MAX_OUT, REL_DIM), lambda i: (i, 0, 0)),
            pl.BlockSpec((BB, MAX_OUT), lambda i: (i, 0)),
            pl.BlockSpec((BB, REL_DIM), lambda i: (i, 0)),
            pl.BlockSpec((BB, MAX_OUT), lambda i: (i, 0)),
        ],
        out_specs=[
            pl.BlockSpec((BB, MAX_OUT), lambda i: (i, 0)),
            pl.BlockSpec((BB, 1), lambda i: (i, 0)),
            pl.BlockSpec((BB, 1), lambda i: (i, 0)),
            pl.BlockSpec((BB, 1), lambda i: (i, 0)),
        ],
        out_shape=(
            jax.ShapeDtypeStruct((B, MAX_OUT), jnp.float32),
            jax.ShapeDtypeStruct((B, 1), jnp.float32),
            jax.ShapeDtypeStruct((B, 1), jnp.int32),
            jax.ShapeDtypeStruct((B, 1), jnp.int32),
        ),
    )(rows, ids, mlp, u)


def kernel(prev_state_h, prev_state_c, prev_relation, actions_id, queries,
           rel_emb, W_ih, W_hh, b_ih, b_hh, W1, b1, W2, b2):
    out_ids = actions_id[:, :, 0]
    ids = jnp.concatenate([
        prev_relation.astype(jnp.int32),
        queries.astype(jnp.int32),
        out_ids.reshape(-1),
        jnp.zeros((N_PAD - N_IDS,), jnp.int32),
    ])
    rows = _sc_gather(rel_emb, ids.reshape(NW, CH, CHUNK))
    prev_emb = rows[:B]
    q_emb = rows[B:2 * B]
    gathered = rows[2 * B:2 * B + B * MAX_OUT].reshape(B, MAX_OUT, REL_DIM)

    h_new, c_new, mlp = _dense(prev_emb, prev_state_h, prev_state_c, q_emb,
                               W_ih, W_hh, b_ih, b_hh, W1, b1, W2, b2)
    u = jnp.asarray(_UNIFORM)
    logits, loss2, act2, cho2 = _score(gathered, out_ids, mlp, u)
    return (loss2[:, 0], logits, h_new, c_new, act2[:, 0], cho2[:, 0])


# speedup vs baseline: 3.8156x; 1.8409x over previous
import functools

import numpy as np
import jax
import jax.numpy as jnp
from jax import lax
from jax.experimental import pallas as pl
from jax.experimental.pallas import tpu as pltpu
from jax.experimental.pallas import tpu_sc as plsc

B = 1024
MAX_OUT = 200
REL_DIM = 64
STATE_DIM = 128
PAD_ID = 0

NW = 32
CHUNK = 128
NCH = (B * MAX_OUT) // (NW * CHUNK)
SMALL = (2 * B) // NW

BB = 64
PK = MAX_OUT // 2


def _uniform_constant():
    def rotl(x, d):
        return (x << np.uint32(d)) | (x >> np.uint32(32 - d))

    with np.errstate(over="ignore"):
        n = np.arange(B * MAX_OUT, dtype=np.uint32)
        ks = [np.uint32(0), np.uint32(1234),
              np.uint32(0) ^ np.uint32(1234) ^ np.uint32(0x1BD11BDA)]
        x = [np.zeros_like(n) + ks[0], n + ks[1]]
        rot0, rot1 = (13, 15, 26, 6), (17, 29, 16, 24)

        def rounds(x, rots):
            for r in rots:
                x[0] = x[0] + x[1]
                x[1] = x[0] ^ rotl(x[1], r)
            return x

        x = rounds(x, rot0); x[0] = x[0] + ks[1]; x[1] = x[1] + ks[2] + np.uint32(1)
        x = rounds(x, rot1); x[0] = x[0] + ks[2]; x[1] = x[1] + ks[0] + np.uint32(2)
        x = rounds(x, rot0); x[0] = x[0] + ks[0]; x[1] = x[1] + ks[1] + np.uint32(3)
        x = rounds(x, rot1); x[0] = x[0] + ks[1]; x[1] = x[1] + ks[2] + np.uint32(4)
        x = rounds(x, rot0); x[0] = x[0] + ks[2]; x[1] = x[1] + ks[0] + np.uint32(5)
        bits = x[0] ^ x[1]

    fb = (bits >> np.uint32(9)) | np.uint32(0x3F800000)
    u = fb.view(np.float32) - np.float32(1.0)
    tiny = np.float32(np.finfo(np.float32).tiny)
    u = np.maximum(tiny, (u * np.float32(1.0 - tiny) + tiny).astype(np.float32))
    return u.reshape(B, MAX_OUT)


_UNIFORM = _uniform_constant()

_SC_MESH = dict(core_axis_name="c", subcore_axis_name="s")


def _sc_gather_small(table, idx2):
    @functools.partial(
        pl.kernel,
        out_type=jax.ShapeDtypeStruct((2 * B, REL_DIM), jnp.float32),
        mesh=plsc.VectorSubcoreMesh(**_SC_MESH),
        compiler_params=pltpu.CompilerParams(use_tc_tiling_on_sc=False),
        scratch_types=[
            pltpu.VMEM((SMALL,), jnp.int32),
            pltpu.VMEM((SMALL, REL_DIM), jnp.float32),
            pltpu.SemaphoreType.DMA,
        ],
    )
    def k(table_hbm, idx_hbm, out_hbm, idx_v, buf, sem):
        wid = lax.axis_index("s") * 2 + lax.axis_index("c")
        pltpu.sync_copy(idx_hbm.at[wid], idx_v)
        pltpu.async_copy(table_hbm.at[idx_v], buf, sem).wait()
        pltpu.sync_copy(buf, out_hbm.at[pl.ds(wid * SMALL, SMALL)])

    return k(table, idx2)


def _sc_gather_big(table, idx3):
    @functools.partial(
        pl.kernel,
        out_type=jax.ShapeDtypeStruct((B * MAX_OUT, REL_DIM), jnp.float32),
        mesh=plsc.VectorSubcoreMesh(**_SC_MESH),
        compiler_params=pltpu.CompilerParams(use_tc_tiling_on_sc=False),
        scratch_types=[
            pltpu.VMEM((NCH, CHUNK), jnp.int32),
            pltpu.VMEM((CHUNK, REL_DIM), jnp.float32),
            pltpu.VMEM((CHUNK, REL_DIM), jnp.float32),
            pltpu.SemaphoreType.DMA,
            pltpu.SemaphoreType.DMA,
        ],
    )
    def k(table_hbm, idx_hbm, out_hbm, idx_v, buf0, buf1, sem0, sem1):
        wid = lax.axis_index("s") * 2 + lax.axis_index("c")
        pltpu.sync_copy(idx_hbm.at[wid], idx_v)
        base = wid * (NCH * CHUNK)

        pltpu.async_copy(table_hbm.at[idx_v.at[0]], buf0, sem0)

        def body(h, carry):
            j = 2 * h
            pltpu.async_copy(table_hbm.at[idx_v.at[j + 1]], buf1, sem1)
            pltpu.make_async_copy(table_hbm.at[idx_v.at[j]], buf0, sem0).wait()
            pltpu.sync_copy(buf0, out_hbm.at[pl.ds(base + j * CHUNK, CHUNK)])

            @pl.when(h < NCH // 2 - 1)
            def _():
                pltpu.async_copy(table_hbm.at[idx_v.at[j + 2]], buf0, sem0)

            pltpu.make_async_copy(table_hbm.at[idx_v.at[j + 1]], buf1, sem1).wait()
            pltpu.sync_copy(buf1, out_hbm.at[pl.ds(base + (j + 1) * CHUNK, CHUNK)])
            return carry

        lax.fori_loop(0, NCH // 2, body, 0)

    return k(table, idx3)


def _dense_body(pq_ref, h_ref, c_ref, wih_ref, whh_ref, bih_ref,
                bhh_ref, w1_ref, b1_ref, w2_ref, b2_ref,
                hout_ref, cout_ref, mlp_ref):
    dn = (((1,), (1,)), ((), ()))
    emb = pq_ref[:, :REL_DIM]
    q = pq_ref[:, REL_DIM:]
    gates = lax.dot_general(emb, wih_ref[...], dn,
                            preferred_element_type=jnp.float32)
    gates = gates + lax.dot_general(h_ref[...], whh_ref[...], dn,
                                    preferred_element_type=jnp.float32)
    gates = gates + bih_ref[...]
    gates = gates + bhh_ref[...]
    i = jax.nn.sigmoid(gates[:, :STATE_DIM])
    f = jax.nn.sigmoid(gates[:, STATE_DIM:2 * STATE_DIM])
    g = jnp.tanh(gates[:, 2 * STATE_DIM:3 * STATE_DIM])
    o = jax.nn.sigmoid(gates[:, 3 * STATE_DIM:])
    c_new = f * c_ref[...] + i * g
    h_new = o * jnp.tanh(c_new)
    hout_ref[...] = h_new
    cout_ref[...] = c_new
    sq = jnp.concatenate([h_new, q], axis=1)
    hid = jnp.maximum(
        lax.dot_general(sq, w1_ref[...], dn,
                        preferred_element_type=jnp.float32) + b1_ref[...], 0.0)
    mlp = jnp.maximum(
        lax.dot_general(hid, w2_ref[...], dn,
                        preferred_element_type=jnp.float32) + b2_ref[...], 0.0)
    mlp_ref[...] = mlp


def _dense(pq, h, c, W_ih, W_hh, b_ih, b_hh, W1, b1, W2, b2):
    return pl.pallas_call(
        _dense_body,
        out_shape=(
            jax.ShapeDtypeStruct((B, STATE_DIM), jnp.float32),
            jax.ShapeDtypeStruct((B, STATE_DIM), jnp.float32),
            jax.ShapeDtypeStruct((B, REL_DIM), jnp.float32),
        ),
    )(pq, h, c, W_ih, W_hh, b_ih.reshape(1, -1),
      b_hh.reshape(1, -1), W1, b1.reshape(1, -1), W2, b2.reshape(1, -1))


def _score_body(pk_ref, ids_ref, mlp_ref, u_ref,
                logits_ref, loss_ref, act_ref, cho_ref):
    pk = pk_ref[...]
    mlp = mlp_ref[...]
    mlpb = jnp.repeat(mlp, PK, axis=0)
    prod = pk * jnp.concatenate([mlpb, mlpb], axis=1)
    se = jnp.sum(prod[:, :REL_DIM], axis=1).reshape(BB, PK)
    so = jnp.sum(prod[:, REL_DIM:], axis=1).reshape(BB, PK)
    scores = jnp.stack([se, so], axis=-1).reshape(BB, MAX_OUT)
    ids = ids_ref[...]
    masked = jnp.where(ids == PAD_ID, jnp.float32(-99999.0), scores)
    gum = -jnp.log(-jnp.log(u_ref[...]))
    z = gum + masked
    zmax = jnp.max(z, axis=1, keepdims=True)
    miota = lax.broadcasted_iota(jnp.int32, (BB, MAX_OUT), 1)
    act = jnp.min(jnp.where(z == zmax, miota, MAX_OUT), axis=1, keepdims=True)
    act_ref[...] = act
    cho_ref[...] = jnp.sum(jnp.where(miota == act, ids, 0),
                           axis=1, keepdims=True)
    smax = jnp.max(masked, axis=1, keepdims=True)
    sh = masked - smax
    lse = jnp.log(jnp.sum(jnp.exp(sh), axis=1, keepdims=True))
    lg = sh - lse
    logits_ref[...] = lg
    loss_ref[...] = -jnp.sum(jnp.where(miota == act, lg, 0.0),
                             axis=1, keepdims=True)


def _score(packed, ids, mlp, u):
    grid = (B // BB,)
    return pl.pallas_call(
        _score_body,
        grid=grid,
        in_specs=[
            pl.BlockSpec((BB * PK, 2 * REL_DIM), lambda i: (i, 0)),
            pl.BlockSpec((BB, MAX_OUT), lambda i: (i, 0)),
            pl.BlockSpec((BB, REL_DIM), lambda i: (i, 0)),
            pl.BlockSpec((BB, MAX_OUT), lambda i: (i, 0)),
        ],
        out_specs=[
            pl.BlockSpec((BB, MAX_OUT), lambda i: (i, 0)),
            pl.BlockSpec((BB, 1), lambda i: (i, 0)),
            pl.BlockSpec((BB, 1), lambda i: (i, 0)),
            pl.BlockSpec((BB, 1), lambda i: (i, 0)),
        ],
        out_shape=(
            jax.ShapeDtypeStruct((B, MAX_OUT), jnp.float32),
            jax.ShapeDtypeStruct((B, 1), jnp.float32),
            jax.ShapeDtypeStruct((B, 1), jnp.int32),
            jax.ShapeDtypeStruct((B, 1), jnp.int32),
        ),
    )(packed, ids, mlp, u)


def kernel(prev_state_h, prev_state_c, prev_relation, actions_id, queries,
           rel_emb, W_ih, W_hh, b_ih, b_hh, W1, b1, W2, b2):
    out_ids = actions_id[:, :, 0]
    small_ids = jnp.stack(
        [prev_relation.astype(jnp.int32), queries.astype(jnp.int32)],
        axis=1).reshape(NW, SMALL)
    rows_small = _sc_gather_small(rel_emb, small_ids)
    pq = rows_small.reshape(B, 2 * REL_DIM)

    rows_big = _sc_gather_big(rel_emb, out_ids.reshape(NW, NCH, CHUNK))
    packed = rows_big.reshape((B * MAX_OUT) // 2, 2 * REL_DIM)

    h_new, c_new, mlp = _dense(pq, prev_state_h, prev_state_c,
                               W_ih, W_hh, b_ih, b_hh, W1, b1, W2, b2)
    u = jnp.asarray(_UNIFORM)
    logits, loss2, act2, cho2 = _score(packed, out_ids, mlp, u)
    return (loss2[:, 0], logits, h_new, c_new, act2[:, 0], cho2[:, 0])


# speedup vs baseline: 3.8246x; 1.0024x over previous
import functools

import numpy as np
import jax
import jax.numpy as jnp
from jax import lax
from jax.experimental import pallas as pl
from jax.experimental.pallas import tpu as pltpu
from jax.experimental.pallas import tpu_sc as plsc

B = 1024
MAX_OUT = 200
REL_DIM = 64
STATE_DIM = 128
PAD_ID = 0

NW = 32
CHUNK = 128
NCH = (B * MAX_OUT) // (NW * CHUNK)
SMALL = (2 * B) // NW

BB = 64
PK = MAX_OUT // 2


def _uniform_constant():
    def rotl(x, d):
        return (x << np.uint32(d)) | (x >> np.uint32(32 - d))

    with np.errstate(over="ignore"):
        n = np.arange(B * MAX_OUT, dtype=np.uint32)
        ks = [np.uint32(0), np.uint32(1234),
              np.uint32(0) ^ np.uint32(1234) ^ np.uint32(0x1BD11BDA)]
        x = [np.zeros_like(n) + ks[0], n + ks[1]]
        rot0, rot1 = (13, 15, 26, 6), (17, 29, 16, 24)

        def rounds(x, rots):
            for r in rots:
                x[0] = x[0] + x[1]
                x[1] = x[0] ^ rotl(x[1], r)
            return x

        x = rounds(x, rot0); x[0] = x[0] + ks[1]; x[1] = x[1] + ks[2] + np.uint32(1)
        x = rounds(x, rot1); x[0] = x[0] + ks[2]; x[1] = x[1] + ks[0] + np.uint32(2)
        x = rounds(x, rot0); x[0] = x[0] + ks[0]; x[1] = x[1] + ks[1] + np.uint32(3)
        x = rounds(x, rot1); x[0] = x[0] + ks[1]; x[1] = x[1] + ks[2] + np.uint32(4)
        x = rounds(x, rot0); x[0] = x[0] + ks[2]; x[1] = x[1] + ks[0] + np.uint32(5)
        bits = x[0] ^ x[1]

    fb = (bits >> np.uint32(9)) | np.uint32(0x3F800000)
    u = fb.view(np.float32) - np.float32(1.0)
    tiny = np.float32(np.finfo(np.float32).tiny)
    u = np.maximum(tiny, (u * np.float32(1.0 - tiny) + tiny).astype(np.float32))
    return u.reshape(B, MAX_OUT)


_UNIFORM = _uniform_constant()

_SC_MESH = dict(core_axis_name="c", subcore_axis_name="s")


def _sc_gather_small(table, idx2):
    @functools.partial(
        pl.kernel,
        out_type=jax.ShapeDtypeStruct((2 * B, REL_DIM), jnp.float32),
        mesh=plsc.VectorSubcoreMesh(**_SC_MESH),
        compiler_params=pltpu.CompilerParams(use_tc_tiling_on_sc=False),
        scratch_types=[
            pltpu.VMEM((SMALL,), jnp.int32),
            pltpu.VMEM((SMALL, REL_DIM), jnp.float32),
            pltpu.SemaphoreType.DMA,
        ],
    )
    def k(table_hbm, idx_hbm, out_hbm, idx_v, buf, sem):
        wid = lax.axis_index("s") * 2 + lax.axis_index("c")
        pltpu.sync_copy(idx_hbm.at[wid], idx_v)
        pltpu.async_copy(table_hbm.at[idx_v], buf, sem).wait()
        pltpu.sync_copy(buf, out_hbm.at[pl.ds(wid * SMALL, SMALL)])

    return k(table, idx2)


def _sc_gather_big(table, idx3):
    @functools.partial(
        pl.kernel,
        out_type=jax.ShapeDtypeStruct((B * MAX_OUT, REL_DIM), jnp.float32),
        mesh=plsc.VectorSubcoreMesh(**_SC_MESH),
        compiler_params=pltpu.CompilerParams(use_tc_tiling_on_sc=False),
        scratch_types=[
            pltpu.VMEM((NCH, CHUNK), jnp.int32),
            pltpu.VMEM((CHUNK, REL_DIM), jnp.float32),
            pltpu.VMEM((CHUNK, REL_DIM), jnp.float32),
            pltpu.SemaphoreType.DMA,
            pltpu.SemaphoreType.DMA,
        ],
    )
    def k(table_hbm, idx_hbm, out_hbm, idx_v, buf0, buf1, sem0, sem1):
        wid = lax.axis_index("s") * 2 + lax.axis_index("c")
        pltpu.sync_copy(idx_hbm.at[wid], idx_v)
        base = wid * (NCH * CHUNK)

        pltpu.async_copy(table_hbm.at[idx_v.at[0]], buf0, sem0)

        def body(h, carry):
            j = 2 * h
            pltpu.async_copy(table_hbm.at[idx_v.at[j + 1]], buf1, sem1)
            pltpu.make_async_copy(table_hbm.at[idx_v.at[j]], buf0, sem0).wait()
            pltpu.sync_copy(buf0, out_hbm.at[pl.ds(base + j * CHUNK, CHUNK)])

            @pl.when(h < NCH // 2 - 1)
            def _():
                pltpu.async_copy(table_hbm.at[idx_v.at[j + 2]], buf0, sem0)

            pltpu.make_async_copy(table_hbm.at[idx_v.at[j + 1]], buf1, sem1).wait()
            pltpu.sync_copy(buf1, out_hbm.at[pl.ds(base + (j + 1) * CHUNK, CHUNK)])
            return carry

        lax.fori_loop(0, NCH // 2, body, 0)

    return k(table, idx3)


def _dense_body(pq_ref, h_ref, c_ref, wih_ref, whh_ref, bih_ref,
                bhh_ref, w1_ref, b1_ref, w2_ref, b2_ref,
                hout_ref, cout_ref, mlp_ref):
    dn = (((1,), (1,)), ((), ()))
    emb = pq_ref[:, :REL_DIM]
    q = pq_ref[:, REL_DIM:]
    gates = lax.dot_general(emb, wih_ref[...], dn,
                            preferred_element_type=jnp.float32)
    gates = gates + lax.dot_general(h_ref[...], whh_ref[...], dn,
                                    preferred_element_type=jnp.float32)
    gates = gates + bih_ref[...]
    gates = gates + bhh_ref[...]
    i = jax.nn.sigmoid(gates[:, :STATE_DIM])
    f = jax.nn.sigmoid(gates[:, STATE_DIM:2 * STATE_DIM])
    g = jnp.tanh(gates[:, 2 * STATE_DIM:3 * STATE_DIM])
    o = jax.nn.sigmoid(gates[:, 3 * STATE_DIM:])
    c_new = f * c_ref[...] + i * g
    h_new = o * jnp.tanh(c_new)
    hout_ref[...] = h_new
    cout_ref[...] = c_new
    sq = jnp.concatenate([h_new, q], axis=1)
    hid = jnp.maximum(
        lax.dot_general(sq, w1_ref[...], dn,
                        preferred_element_type=jnp.float32) + b1_ref[...], 0.0)
    mlp = jnp.maximum(
        lax.dot_general(hid, w2_ref[...], dn,
                        preferred_element_type=jnp.float32) + b2_ref[...], 0.0)
    mlp_ref[...] = mlp


def _dense(pq, h, c, W_ih, W_hh, b_ih, b_hh, W1, b1, W2, b2):
    return pl.pallas_call(
        _dense_body,
        out_shape=(
            jax.ShapeDtypeStruct((B, STATE_DIM), jnp.float32),
            jax.ShapeDtypeStruct((B, STATE_DIM), jnp.float32),
            jax.ShapeDtypeStruct((B, REL_DIM), jnp.float32),
        ),
    )(pq, h, c, W_ih, W_hh, b_ih.reshape(1, -1),
      b_hh.reshape(1, -1), W1, b1.reshape(1, -1), W2, b2.reshape(1, -1))


def _score_body(pk_ref, ids_ref, mlp_ref, u_ref,
                logits_ref, loss_ref, act_ref, cho_ref):
    pk = pk_ref[...]
    mlp = mlp_ref[...]
    mlpb = jnp.repeat(mlp, PK, axis=0)
    prod = pk * jnp.concatenate([mlpb, mlpb], axis=1)
    se = jnp.sum(prod[:, :REL_DIM], axis=1).reshape(BB, PK)
    so = jnp.sum(prod[:, REL_DIM:], axis=1).reshape(BB, PK)
    scores = jnp.stack([se, so], axis=-1).reshape(BB, MAX_OUT)
    ids = ids_ref[...]
    masked = jnp.where(ids == PAD_ID, jnp.float32(-99999.0), scores)
    gum = -jnp.log(-jnp.log(u_ref[...]))
    z = gum + masked
    zmax = jnp.max(z, axis=1, keepdims=True)
    miota = lax.broadcasted_iota(jnp.int32, (BB, MAX_OUT), 1)
    act = jnp.min(jnp.where(z == zmax, miota, MAX_OUT), axis=1, keepdims=True)
    act_ref[...] = act
    cho_ref[...] = jnp.sum(jnp.where(miota == act, ids, 0),
                           axis=1, keepdims=True)
    smax = jnp.max(masked, axis=1, keepdims=True)
    sh = masked - smax
    lse = jnp.log(jnp.sum(jnp.exp(sh), axis=1, keepdims=True))
    lg = sh - lse
    logits_ref[...] = lg
    loss_ref[...] = -jnp.sum(jnp.where(miota == act, lg, 0.0),
                             axis=1, keepdims=True)


def _score(packed, ids, mlp, u):
    grid = (B // BB,)
    return pl.pallas_call(
        _score_body,
        grid=grid,
        in_specs=[
            pl.BlockSpec((BB * PK, 2 * REL_DIM), lambda i: (i, 0)),
            pl.BlockSpec((BB, MAX_OUT), lambda i: (i, 0)),
            pl.BlockSpec((BB, REL_DIM), lambda i: (i, 0)),
            pl.BlockSpec((BB, MAX_OUT), lambda i: (i, 0)),
        ],
        out_specs=[
            pl.BlockSpec((BB, MAX_OUT), lambda i: (i, 0)),
            pl.BlockSpec((BB, 1), lambda i: (i, 0)),
            pl.BlockSpec((BB, 1), lambda i: (i, 0)),
            pl.BlockSpec((BB, 1), lambda i: (i, 0)),
        ],
        out_shape=(
            jax.ShapeDtypeStruct((B, MAX_OUT), jnp.float32),
            jax.ShapeDtypeStruct((B, 1), jnp.float32),
            jax.ShapeDtypeStruct((B, 1), jnp.int32),
            jax.ShapeDtypeStruct((B, 1), jnp.int32),
        ),
    )(packed, ids, mlp, u)


def kernel(prev_state_h, prev_state_c, prev_relation, actions_id, queries,
           rel_emb, W_ih, W_hh, b_ih, b_hh, W1, b1, W2, b2):
    out_ids = actions_id[:, :, 0]
    rel_lin = lax.optimization_barrier(rel_emb.reshape(-1))
    tbl = rel_lin.reshape(rel_emb.shape)
    small_ids = jnp.stack(
        [prev_relation.astype(jnp.int32), queries.astype(jnp.int32)],
        axis=1).reshape(NW, SMALL)
    rows_small = _sc_gather_small(tbl, small_ids)
    pq = rows_small.reshape(B, 2 * REL_DIM)

    rows_big = _sc_gather_big(tbl, out_ids.reshape(NW, NCH, CHUNK))
    packed = rows_big.reshape((B * MAX_OUT) // 2, 2 * REL_DIM)

    h_new, c_new, mlp = _dense(pq, prev_state_h, prev_state_c,
                               W_ih, W_hh, b_ih, b_hh, W1, b1, W2, b2)
    u = jnp.asarray(_UNIFORM)
    logits, loss2, act2, cho2 = _score(packed, out_ids, mlp, u)
    return (loss2[:, 0], logits, h_new, c_new, act2[:, 0], cho2[:, 0])


# speedup vs baseline: 4.1521x; 1.0856x over previous
import functools

import numpy as np
import jax
import jax.numpy as jnp
from jax import lax
from jax.experimental import pallas as pl
from jax.experimental.pallas import tpu as pltpu
from jax.experimental.pallas import tpu_sc as plsc

B = 1024
MAX_OUT = 200
REL_DIM = 64
STATE_DIM = 128
PAD_ID = 0

NW = 32
CHUNK = 128
HALF = B // 2
NCH = (HALF * MAX_OUT) // (NW * CHUNK)
SMALL = (2 * B) // NW

BB = 64
PK = MAX_OUT // 2


def _uniform_constant():
    def rotl(x, d):
        return (x << np.uint32(d)) | (x >> np.uint32(32 - d))

    with np.errstate(over="ignore"):
        n = np.arange(B * MAX_OUT, dtype=np.uint32)
        ks = [np.uint32(0), np.uint32(1234),
              np.uint32(0) ^ np.uint32(1234) ^ np.uint32(0x1BD11BDA)]
        x = [np.zeros_like(n) + ks[0], n + ks[1]]
        rot0, rot1 = (13, 15, 26, 6), (17, 29, 16, 24)

        def rounds(x, rots):
            for r in rots:
                x[0] = x[0] + x[1]
                x[1] = x[0] ^ rotl(x[1], r)
            return x

        x = rounds(x, rot0); x[0] = x[0] + ks[1]; x[1] = x[1] + ks[2] + np.uint32(1)
        x = rounds(x, rot1); x[0] = x[0] + ks[2]; x[1] = x[1] + ks[0] + np.uint32(2)
        x = rounds(x, rot0); x[0] = x[0] + ks[0]; x[1] = x[1] + ks[1] + np.uint32(3)
        x = rounds(x, rot1); x[0] = x[0] + ks[1]; x[1] = x[1] + ks[2] + np.uint32(4)
        x = rounds(x, rot0); x[0] = x[0] + ks[2]; x[1] = x[1] + ks[0] + np.uint32(5)
        bits = x[0] ^ x[1]

    fb = (bits >> np.uint32(9)) | np.uint32(0x3F800000)
    u = fb.view(np.float32) - np.float32(1.0)
    tiny = np.float32(np.finfo(np.float32).tiny)
    u = np.maximum(tiny, (u * np.float32(1.0 - tiny) + tiny).astype(np.float32))
    return u.reshape(B, MAX_OUT)


_UNIFORM = _uniform_constant()

_SC_MESH = dict(core_axis_name="c", subcore_axis_name="s")


def _sc_gather_small(table, idx2):
    @functools.partial(
        pl.kernel,
        out_type=jax.ShapeDtypeStruct((2 * B, REL_DIM), jnp.float32),
        mesh=plsc.VectorSubcoreMesh(**_SC_MESH),
        compiler_params=pltpu.CompilerParams(use_tc_tiling_on_sc=False),
        scratch_types=[
            pltpu.VMEM((SMALL,), jnp.int32),
            pltpu.VMEM((SMALL, REL_DIM), jnp.float32),
            pltpu.SemaphoreType.DMA,
        ],
    )
    def k(table_hbm, idx_hbm, out_hbm, idx_v, buf, sem):
        wid = lax.axis_index("s") * 2 + lax.axis_index("c")
        pltpu.sync_copy(idx_hbm.at[wid], idx_v)
        pltpu.async_copy(table_hbm.at[idx_v], buf, sem).wait()
        pltpu.sync_copy(buf, out_hbm.at[pl.ds(wid * SMALL, SMALL)])

    return k(table, idx2)


def _sc_gather_big(table, idx3):
    @functools.partial(
        pl.kernel,
        out_type=jax.ShapeDtypeStruct((HALF * MAX_OUT, REL_DIM), jnp.float32),
        mesh=plsc.VectorSubcoreMesh(**_SC_MESH),
        compiler_params=pltpu.CompilerParams(use_tc_tiling_on_sc=False),
        scratch_types=[
            pltpu.VMEM((NCH, CHUNK), jnp.int32),
            pltpu.VMEM((CHUNK, REL_DIM), jnp.float32),
            pltpu.VMEM((CHUNK, REL_DIM), jnp.float32),
            pltpu.SemaphoreType.DMA,
            pltpu.SemaphoreType.DMA,
        ],
    )
    def k(table_hbm, idx_hbm, out_hbm, idx_v, buf0, buf1, sem0, sem1):
        wid = lax.axis_index("s") * 2 + lax.axis_index("c")
        pltpu.sync_copy(idx_hbm.at[wid], idx_v)
        base = wid * (NCH * CHUNK)

        pltpu.async_copy(table_hbm.at[idx_v.at[0]], buf0, sem0)

        def body(h, carry):
            j = 2 * h
            pltpu.async_copy(table_hbm.at[idx_v.at[j + 1]], buf1, sem1)
            pltpu.make_async_copy(table_hbm.at[idx_v.at[j]], buf0, sem0).wait()
            pltpu.sync_copy(buf0, out_hbm.at[pl.ds(base + j * CHUNK, CHUNK)])

            @pl.when(h < NCH // 2 - 1)
            def _():
                pltpu.async_copy(table_hbm.at[idx_v.at[j + 2]], buf0, sem0)

            pltpu.make_async_copy(table_hbm.at[idx_v.at[j + 1]], buf1, sem1).wait()
            pltpu.sync_copy(buf1, out_hbm.at[pl.ds(base + (j + 1) * CHUNK, CHUNK)])
            return carry

        lax.fori_loop(0, NCH // 2, body, 0)

    return k(table, idx3)


def _dense_body(pq_ref, h_ref, c_ref, wih_ref, whh_ref, bih_ref,
                bhh_ref, w1_ref, b1_ref, w2_ref, b2_ref,
                hout_ref, cout_ref, mlp_ref):
    dn = (((1,), (1,)), ((), ()))
    emb = pq_ref[:, :REL_DIM]
    q = pq_ref[:, REL_DIM:]
    gates = lax.dot_general(emb, wih_ref[...], dn,
                            preferred_element_type=jnp.float32)
    gates = gates + lax.dot_general(h_ref[...], whh_ref[...], dn,
                                    preferred_element_type=jnp.float32)
    gates = gates + bih_ref[...]
    gates = gates + bhh_ref[...]
    i = jax.nn.sigmoid(gates[:, :STATE_DIM])
    f = jax.nn.sigmoid(gates[:, STATE_DIM:2 * STATE_DIM])
    g = jnp.tanh(gates[:, 2 * STATE_DIM:3 * STATE_DIM])
    o = jax.nn.sigmoid(gates[:, 3 * STATE_DIM:])
    c_new = f * c_ref[...] + i * g
    h_new = o * jnp.tanh(c_new)
    hout_ref[...] = h_new
    cout_ref[...] = c_new
    sq = jnp.concatenate([h_new, q], axis=1)
    hid = jnp.maximum(
        lax.dot_general(sq, w1_ref[...], dn,
                        preferred_element_type=jnp.float32) + b1_ref[...], 0.0)
    mlp = jnp.maximum(
        lax.dot_general(hid, w2_ref[...], dn,
                        preferred_element_type=jnp.float32) + b2_ref[...], 0.0)
    mlp_ref[...] = mlp


def _dense(pq, h, c, W_ih, W_hh, b_ih, b_hh, W1, b1, W2, b2):
    return pl.pallas_call(
        _dense_body,
        out_shape=(
            jax.ShapeDtypeStruct((B, STATE_DIM), jnp.float32),
            jax.ShapeDtypeStruct((B, STATE_DIM), jnp.float32),
            jax.ShapeDtypeStruct((B, REL_DIM), jnp.float32),
        ),
    )(pq, h, c, W_ih, W_hh, b_ih.reshape(1, -1),
      b_hh.reshape(1, -1), W1, b1.reshape(1, -1), W2, b2.reshape(1, -1))


def _score_body(pk_ref, ids_ref, mlp_ref, u_ref,
                logits_ref, loss_ref, act_ref, cho_ref):
    pk = pk_ref[...]
    mlp = mlp_ref[...]
    mlpb = jnp.repeat(mlp, PK, axis=0)
    prod = pk * jnp.concatenate([mlpb, mlpb], axis=1)
    se = jnp.sum(prod[:, :REL_DIM], axis=1).reshape(BB, PK)
    so = jnp.sum(prod[:, REL_DIM:], axis=1).reshape(BB, PK)
    scores = jnp.stack([se, so], axis=-1).reshape(BB, MAX_OUT)
    ids = ids_ref[...]
    masked = jnp.where(ids == PAD_ID, jnp.float32(-99999.0), scores)
    gum = -jnp.log(-jnp.log(u_ref[...]))
    z = gum + masked
    zmax = jnp.max(z, axis=1, keepdims=True)
    miota = lax.broadcasted_iota(jnp.int32, (BB, MAX_OUT), 1)
    act = jnp.min(jnp.where(z == zmax, miota, MAX_OUT), axis=1, keepdims=True)
    act_ref[...] = act
    cho_ref[...] = jnp.sum(jnp.where(miota == act, ids, 0),
                           axis=1, keepdims=True)
    smax = jnp.max(masked, axis=1, keepdims=True)
    sh = masked - smax
    lse = jnp.log(jnp.sum(jnp.exp(sh), axis=1, keepdims=True))
    lg = sh - lse
    logits_ref[...] = lg
    loss_ref[...] = -jnp.sum(jnp.where(miota == act, lg, 0.0),
                             axis=1, keepdims=True)


def _score(packed, ids, mlp, u):
    grid = (HALF // BB,)
    return pl.pallas_call(
        _score_body,
        grid=grid,
        in_specs=[
            pl.BlockSpec((BB * PK, 2 * REL_DIM), lambda i: (i, 0)),
            pl.BlockSpec((BB, MAX_OUT), lambda i: (i, 0)),
            pl.BlockSpec((BB, REL_DIM), lambda i: (i, 0)),
            pl.BlockSpec((BB, MAX_OUT), lambda i: (i, 0)),
        ],
        out_specs=[
            pl.BlockSpec((BB, MAX_OUT), lambda i: (i, 0)),
            pl.BlockSpec((BB, 1), lambda i: (i, 0)),
            pl.BlockSpec((BB, 1), lambda i: (i, 0)),
            pl.BlockSpec((BB, 1), lambda i: (i, 0)),
        ],
        out_shape=(
            jax.ShapeDtypeStruct((HALF, MAX_OUT), jnp.float32),
            jax.ShapeDtypeStruct((HALF, 1), jnp.float32),
            jax.ShapeDtypeStruct((HALF, 1), jnp.int32),
            jax.ShapeDtypeStruct((HALF, 1), jnp.int32),
        ),
    )(packed, ids, mlp, u)


def kernel(prev_state_h, prev_state_c, prev_relation, actions_id, queries,
           rel_emb, W_ih, W_hh, b_ih, b_hh, W1, b1, W2, b2):
    out_ids = actions_id[:, :, 0]
    rel_lin = lax.optimization_barrier(rel_emb.reshape(-1))
    tbl = rel_lin.reshape(rel_emb.shape)
    small_ids = jnp.stack(
        [prev_relation.astype(jnp.int32), queries.astype(jnp.int32)],
        axis=1).reshape(NW, SMALL)
    rows_small = _sc_gather_small(tbl, small_ids)
    pq = rows_small.reshape(B, 2 * REL_DIM)

    h_new, c_new, mlp = _dense(pq, prev_state_h, prev_state_c,
                               W_ih, W_hh, b_ih, b_hh, W1, b1, W2, b2)
    u = jnp.asarray(_UNIFORM)

    halves = []
    for hh in range(2):
        ids_h = lax.slice_in_dim(out_ids, hh * HALF, (hh + 1) * HALF, axis=0)
        rows_h = _sc_gather_big(tbl, ids_h.reshape(NW, NCH, CHUNK))
        packed_h = rows_h.reshape((HALF * MAX_OUT) // 2, 2 * REL_DIM)
        mlp_h = lax.slice_in_dim(mlp, hh * HALF, (hh + 1) * HALF, axis=0)
        u_h = lax.slice_in_dim(u, hh * HALF, (hh + 1) * HALF, axis=0)
        halves.append(_score(packed_h, ids_h, mlp_h, u_h))
    logits = jnp.concatenate([halves[0][0], halves[1][0]], axis=0)
    loss2 = jnp.concatenate([halves[0][1], halves[1][1]], axis=0)
    act2 = jnp.concatenate([halves[0][2], halves[1][2]], axis=0)
    cho2 = jnp.concatenate([halves[0][3], halves[1][3]], axis=0)
    return (loss2[:, 0], logits, h_new, c_new, act2[:, 0], cho2[:, 0])
